# Initial kernel scaffold; baseline (speedup 1.0000x reference)
import jax
import jax.numpy as jnp
from jax.experimental import pallas as pl


def kernel(x, table, gamma, beta, W, b):
    raise NotImplementedError("write your pallas kernel here")


# speedup vs baseline: 2.5608x; 2.5608x over previous
import functools

import jax
import jax.numpy as jnp
from jax import lax
from jax.experimental import pallas as pl
from jax.experimental.pallas import tpu as pltpu
from jax.experimental.pallas import tpu_sc as plsc

BATCH = 16384
HIST = 50
EMBED = 32
EPS = 1e-5

NC = 2
NS = 16
NW = NC * NS
RW = BATCH // NW
CROWS = 2
CIDX = CROWS * HIST
NCHUNK = RW // CROWS


def _reduce_chunk(gbuf, acc, c):
    for r in range(CROWS):
        base = r * HIST
        for half in range(2):
            col = pl.ds(half * 16, 16)
            chains = []
            for k in range(4):
                t = gbuf[base + k, col]
                j = base + k + 4
                while j < base + HIST:
                    t = t + gbuf[j, col]
                    j += 4
                chains.append(t)
            acc[c * CROWS + r, col] = (chains[0] + chains[1]) + (
                chains[2] + chains[3])


@functools.partial(
    pl.kernel,
    mesh=plsc.VectorSubcoreMesh(core_axis_name="c", subcore_axis_name="s"),
    out_type=jax.ShapeDtypeStruct((BATCH, EMBED), jnp.float32),
    compiler_params=pltpu.CompilerParams(use_tc_tiling_on_sc=False),
    scratch_types=[
        pltpu.VMEM((NCHUNK, CIDX), jnp.int32),
        pltpu.VMEM((CIDX, EMBED), jnp.float32),
        pltpu.VMEM((CIDX, EMBED), jnp.float32),
        pltpu.VMEM((RW, EMBED), jnp.float32),
        pltpu.SemaphoreType.DMA,
        pltpu.SemaphoreType.DMA,
    ],
)
def _sc_pool(x_hbm, table_hbm, out_hbm, idx_v, gbuf0, gbuf1, acc, sem0, sem1):
    wid = lax.axis_index("s") * NC + lax.axis_index("c")
    pltpu.sync_copy(x_hbm.at[pl.ds(wid * NCHUNK, NCHUNK)], idx_v)

    def gather(c, gbuf, sem):
        return pltpu.make_async_copy(table_hbm.at[idx_v.at[c]], gbuf, sem)

    gather(0, gbuf0, sem0).start()

    def step(g, carry):
        c0 = g * 2
        gather(c0 + 1, gbuf1, sem1).start()
        gather(c0, gbuf0, sem0).wait()
        _reduce_chunk(gbuf0, acc, c0)

        @pl.when(c0 + 2 < NCHUNK)
        def _():
            gather(c0 + 2, gbuf0, sem0).start()

        gather(c0 + 1, gbuf1, sem1).wait()
        _reduce_chunk(gbuf1, acc, c0 + 1)
        return carry

    lax.fori_loop(0, NCHUNK // 2, step, 0)
    pltpu.sync_copy(acc, out_hbm.at[pl.ds(wid * RW, RW)])


def _head_body(s_ref, g_ref, be_ref, w_ref, b_ref, o_ref):
    s = s_ref[...]
    mean_s = jnp.mean(s, axis=0, keepdims=True)
    d = s - mean_s
    var_s = jnp.mean(d * d, axis=0, keepdims=True)
    g = g_ref[...]
    w = w_ref[...]
    gm, gs = g[:, :EMBED], g[:, EMBED:]
    wm, ws = w[:, :EMBED], w[:, EMBED:]
    inv_m = lax.rsqrt(var_s * (1.0 / (HIST * HIST)) + EPS)
    inv_s = lax.rsqrt(var_s + EPS)
    v = gm * inv_m * (1.0 / HIST) * wm + gs * inv_s * ws
    const = jnp.sum(be_ref[...] * w) + b_ref[0, 0] - jnp.sum(mean_s * v)
    logit = jnp.sum(s * v, axis=1, keepdims=True) + const
    o_ref[...] = 1.0 / (1.0 + jnp.exp(-logit))


def _tc_head(s, gamma, beta, W, b):
    return pl.pallas_call(
        _head_body,
        out_shape=jax.ShapeDtypeStruct((BATCH, 1), jnp.float32),
    )(s, gamma, beta, W, b)


def kernel(x, table, gamma, beta, W, b):
    x2 = x.reshape(NW * NCHUNK, CIDX).astype(jnp.int32)
    s = _sc_pool(x2, table)
    return _tc_head(
        s,
        gamma.reshape(1, 2 * EMBED),
        beta.reshape(1, 2 * EMBED),
        W.reshape(1, 2 * EMBED),
        b.reshape(1, 1),
    )
